# baseline (speedup 1.0000x reference)
import jax
import jax.numpy as jnp
from jax.experimental import pallas as pl
from jax.experimental.pallas import tpu as pltpu

B, D, H, W, K = 64, 512, 32, 32, 64
N = H * W
DG = 64


def _netvlad_kernel(x_ref, wt_ref, b_ref, ct_ref, o_ref):
    xb = x_ref[0]
    ft = jax.lax.dot_general(
        xb, wt_ref[...], (((0,), (0,)), ((), ())),
        preferred_element_type=jnp.float32)
    ft = ft + b_ref[...]
    f3 = ft.reshape(H, W, K)
    m = jnp.max(f3, axis=0, keepdims=True)
    e3 = jnp.exp(f3 - m)
    s = jnp.sum(e3, axis=0, keepdims=True)
    a = (e3 / s).reshape(N, K)
    asum = jnp.sum(a, axis=0, keepdims=True)
    vt = jax.lax.dot_general(
        xb, a, (((1,), (0,)), ((), ())),
        preferred_element_type=jnp.float32)
    o_ref[0] = vt - asum * ct_ref[...]


def _transpose_norm_kernel(u_ref, o_ref):
    t = jnp.transpose(u_ref[...], (1, 2, 0))
    ss = jnp.sum(t * t, axis=1, keepdims=True)
    o_ref[...] = t * jax.lax.rsqrt(jnp.maximum(ss, 1e-24))


def kernel(x, w, b_conv, c):
    xf = x.reshape(B, D, N)
    wt = w.T
    ct = c.T
    b2 = b_conv.reshape(1, K)
    u = pl.pallas_call(
        _netvlad_kernel,
        grid=(B,),
        in_specs=[
            pl.BlockSpec((1, D, N), lambda i: (i, 0, 0)),
            pl.BlockSpec((D, K), lambda i: (0, 0)),
            pl.BlockSpec((1, K), lambda i: (0, 0)),
            pl.BlockSpec((D, K), lambda i: (0, 0)),
        ],
        out_specs=pl.BlockSpec((1, D, K), lambda i: (i, 0, 0)),
        out_shape=jax.ShapeDtypeStruct((B, D, K), jnp.float32),
        compiler_params=pltpu.CompilerParams(
            dimension_semantics=("arbitrary",),
        ),
    )(xf, wt, b2, ct)
    y = pl.pallas_call(
        _transpose_norm_kernel,
        grid=(D // DG,),
        in_specs=[pl.BlockSpec((B, DG, K), lambda i: (0, i, 0))],
        out_specs=pl.BlockSpec((DG, K, B), lambda i: (i, 0, 0)),
        out_shape=jax.ShapeDtypeStruct((D, K, B), jnp.float32),
        compiler_params=pltpu.CompilerParams(
            dimension_semantics=("arbitrary",),
        ),
    )(u)
    return y


# speedup vs baseline: 2.6323x; 2.6323x over previous
import jax
import jax.numpy as jnp
from jax.experimental import pallas as pl
from jax.experimental.pallas import tpu as pltpu

B, D, H, W, K = 64, 512, 32, 32, 64
N = H * W


def _netvlad_kernel(x_ref, wt_ref, b_ref, c_ref, o_ref, acc_ref):
    xn = x_ref[0]
    ft = jnp.dot(xn, wt_ref[...], preferred_element_type=jnp.float32)
    ft = ft + b_ref[...]
    f3 = ft.reshape(H, W, K)
    m = jnp.max(f3, axis=0, keepdims=True)
    e3 = jnp.exp(f3 - m)
    s = jnp.sum(e3, axis=0, keepdims=True)
    a = (e3 / s).reshape(N, K)
    asum = jnp.sum(a, axis=0, keepdims=True)
    v = jax.lax.dot_general(
        a, xn, (((0,), (0,)), ((), ())),
        preferred_element_type=jnp.float32)
    v = v - asum.T * c_ref[...]
    ss = jnp.sum(v * v, axis=0, keepdims=True)
    y = v * jax.lax.rsqrt(jnp.maximum(ss, 1e-24))
    j = jax.lax.rem(pl.program_id(0), 8)
    acc_ref[pl.ds(j, 1)] = y.reshape(1, K, D)

    @pl.when(j == 7)
    def _():
        o_ref[:, 0, :, :] = jnp.transpose(acc_ref[...], (1, 0, 2))


def kernel(x, w, b_conv, c):
    xn = x.transpose(0, 2, 3, 1).reshape(B, N, D)
    wt = w.T
    b2 = b_conv.reshape(1, K)
    out = pl.pallas_call(
        _netvlad_kernel,
        grid=(B,),
        in_specs=[
            pl.BlockSpec((1, N, D), lambda i: (i, 0, 0)),
            pl.BlockSpec((D, K), lambda i: (0, 0)),
            pl.BlockSpec((1, K), lambda i: (0, 0)),
            pl.BlockSpec((K, D), lambda i: (0, 0)),
        ],
        out_specs=pl.BlockSpec((K, 1, 8, D), lambda i: (0, i // 8, 0, 0)),
        out_shape=jax.ShapeDtypeStruct((K, B // 8, 8, D), jnp.float32),
        scratch_shapes=[pltpu.VMEM((8, K, D), jnp.float32)],
        compiler_params=pltpu.CompilerParams(
            dimension_semantics=("arbitrary",),
        ),
    )(xn, wt, b2, c)
    return jnp.transpose(out.reshape(K, B, D), (2, 0, 1))


# speedup vs baseline: 2.6490x; 1.0063x over previous
import jax
import jax.numpy as jnp
from jax.experimental import pallas as pl
from jax.experimental.pallas import tpu as pltpu

B, D, H, W, K = 64, 512, 32, 32, 64
N = H * W


def _netvlad_kernel(x_ref, wt_ref, b_ref, c_ref, o_ref, acc_ref):
    xn = x_ref[0]
    xb = xn.astype(jnp.bfloat16)
    ft = jnp.dot(xb, wt_ref[...], preferred_element_type=jnp.float32)
    ft = ft + b_ref[...]
    f3 = ft.reshape(H, W, K)
    m = jnp.max(f3, axis=0, keepdims=True)
    e3 = jnp.exp(f3 - m)
    s = jnp.sum(e3, axis=0, keepdims=True)
    a = (e3 / s).reshape(N, K)
    asum = jnp.sum(a, axis=0, keepdims=True)
    v = jax.lax.dot_general(
        a.astype(jnp.bfloat16), xb, (((0,), (0,)), ((), ())),
        preferred_element_type=jnp.float32)
    v = v - asum.T * c_ref[...]
    ss = jnp.sum(v * v, axis=0, keepdims=True)
    y = v * jax.lax.rsqrt(jnp.maximum(ss, 1e-24))
    j = jax.lax.rem(pl.program_id(0), 8)
    acc_ref[pl.ds(j, 1)] = y.reshape(1, K, D)

    @pl.when(j == 7)
    def _():
        o_ref[:, 0, :, :] = jnp.transpose(acc_ref[...], (1, 0, 2))


def kernel(x, w, b_conv, c):
    xn = x.transpose(0, 2, 3, 1).reshape(B, N, D)
    wt = w.T.astype(jnp.bfloat16)
    b2 = b_conv.reshape(1, K)
    out = pl.pallas_call(
        _netvlad_kernel,
        grid=(B,),
        in_specs=[
            pl.BlockSpec((1, N, D), lambda i: (i, 0, 0)),
            pl.BlockSpec((D, K), lambda i: (0, 0)),
            pl.BlockSpec((1, K), lambda i: (0, 0)),
            pl.BlockSpec((K, D), lambda i: (0, 0)),
        ],
        out_specs=pl.BlockSpec((K, 1, 8, D), lambda i: (0, i // 8, 0, 0)),
        out_shape=jax.ShapeDtypeStruct((K, B // 8, 8, D), jnp.float32),
        scratch_shapes=[pltpu.VMEM((8, K, D), jnp.float32)],
        compiler_params=pltpu.CompilerParams(
            dimension_semantics=("arbitrary",),
        ),
    )(xn, wt, b2, c)
    return jnp.transpose(out.reshape(K, B, D), (2, 0, 1))


# speedup vs baseline: 3.1404x; 1.1855x over previous
import jax
import jax.numpy as jnp
from jax.experimental import pallas as pl
from jax.experimental.pallas import tpu as pltpu

B, D, H, W, K = 64, 512, 32, 32, 64
N = H * W
G = 2


def _one_batch(xn, wt_ref, b_ref, cw_ref):
    xb = xn.astype(jnp.bfloat16)
    ft = jnp.dot(xb, wt_ref[...], preferred_element_type=jnp.float32)
    ft = ft + b_ref[...]
    e3 = jnp.exp(ft.reshape(H, W, K))
    s = jnp.sum(e3, axis=0, keepdims=True)
    a = (e3 / s).reshape(N, K)
    v = jax.lax.dot_general(
        a.astype(jnp.bfloat16), xb, (((0,), (0,)), ((), ())),
        preferred_element_type=jnp.float32)
    v = v - cw_ref[...]
    ss = jnp.sum(v * v, axis=0, keepdims=True)
    return v * jax.lax.rsqrt(jnp.maximum(ss, 1e-24))


def _netvlad_kernel(x_ref, wt_ref, b_ref, cw_ref, o_ref, acc_ref):
    i = pl.program_id(0)
    j = jax.lax.rem(i, 4) * G
    for g in range(G):
        y = _one_batch(x_ref[0, g], wt_ref, b_ref, cw_ref)
        acc_ref[pl.ds(j + g, 1)] = y.reshape(1, K, D)

    @pl.when(jax.lax.rem(i, 4) == 3)
    def _():
        o_ref[:, 0, :, :] = jnp.transpose(acc_ref[...], (1, 0, 2))


def kernel(x, w, b_conv, c):
    xn = x.transpose(0, 2, 3, 1).reshape(B // G, G, N, D)
    wt = w.T.astype(jnp.bfloat16)
    b2 = b_conv.reshape(1, K)
    cw = c * jnp.float32(W)
    out = pl.pallas_call(
        _netvlad_kernel,
        grid=(B // G,),
        in_specs=[
            pl.BlockSpec((1, G, N, D), lambda i: (i, 0, 0, 0)),
            pl.BlockSpec((D, K), lambda i: (0, 0)),
            pl.BlockSpec((1, K), lambda i: (0, 0)),
            pl.BlockSpec((K, D), lambda i: (0, 0)),
        ],
        out_specs=pl.BlockSpec((K, 1, 8, D), lambda i: (0, i // 4, 0, 0)),
        out_shape=jax.ShapeDtypeStruct((K, B // 8, 8, D), jnp.float32),
        scratch_shapes=[pltpu.VMEM((8, K, D), jnp.float32)],
        compiler_params=pltpu.CompilerParams(
            dimension_semantics=("arbitrary",),
        ),
    )(xn, wt, b2, cw)
    return jnp.transpose(out.reshape(K, B, D), (2, 0, 1))


# speedup vs baseline: 3.3271x; 1.0595x over previous
import jax
import jax.numpy as jnp
from jax.experimental import pallas as pl
from jax.experimental.pallas import tpu as pltpu

B, D, H, W, K = 64, 512, 32, 32, 64
N = H * W
G = 4


def _one_batch(xn, wt_ref, b_ref, cw_ref):
    xb = xn.astype(jnp.bfloat16)
    ft = jnp.dot(xb, wt_ref[...], preferred_element_type=jnp.float32)
    ft = ft + b_ref[...]
    e3 = jnp.exp(ft.reshape(H, W, K))
    s = jnp.sum(e3, axis=0, keepdims=True)
    a = (e3 / s).reshape(N, K)
    v = jax.lax.dot_general(
        a.astype(jnp.bfloat16), xb, (((0,), (0,)), ((), ())),
        preferred_element_type=jnp.float32)
    v = v - cw_ref[...]
    ss = jnp.sum(v * v, axis=0, keepdims=True)
    return v * jax.lax.rsqrt(jnp.maximum(ss, 1e-24))


def _netvlad_kernel(x_ref, wt_ref, b_ref, cw_ref, o_ref, acc_ref):
    i = pl.program_id(0)
    j = jax.lax.rem(i, 8 // G) * G
    for g in range(G):
        y = _one_batch(x_ref[0, g], wt_ref, b_ref, cw_ref)
        acc_ref[pl.ds(j + g, 1)] = y.reshape(1, K, D)

    @pl.when(jax.lax.rem(i, 8 // G) == 8 // G - 1)
    def _():
        o_ref[:, 0, :, :] = jnp.transpose(acc_ref[...], (1, 0, 2))


def kernel(x, w, b_conv, c):
    xn = x.transpose(0, 2, 3, 1).reshape(B // G, G, N, D)
    wt = w.T.astype(jnp.bfloat16)
    b2 = b_conv.reshape(1, K)
    cw = c * jnp.float32(W)
    out = pl.pallas_call(
        _netvlad_kernel,
        grid=(B // G,),
        in_specs=[
            pl.BlockSpec((1, G, N, D), lambda i: (i, 0, 0, 0)),
            pl.BlockSpec((D, K), lambda i: (0, 0)),
            pl.BlockSpec((1, K), lambda i: (0, 0)),
            pl.BlockSpec((K, D), lambda i: (0, 0)),
        ],
        out_specs=pl.BlockSpec((K, 1, 8, D), lambda i: (0, i // (8 // G), 0, 0)),
        out_shape=jax.ShapeDtypeStruct((K, B // 8, 8, D), jnp.float32),
        scratch_shapes=[pltpu.VMEM((8, K, D), jnp.float32)],
        compiler_params=pltpu.CompilerParams(
            dimension_semantics=("arbitrary",),
        ),
    )(xn, wt, b2, cw)
    return jnp.transpose(out.reshape(K, B, D), (2, 0, 1))


# speedup vs baseline: 3.3775x; 1.0151x over previous
import jax
import jax.numpy as jnp
from jax.experimental import pallas as pl
from jax.experimental.pallas import tpu as pltpu

B, D, H, W, K = 64, 512, 32, 32, 64
N = H * W
G = 8


def _one_batch(xn, wt_ref, cw_ref):
    xb = xn.astype(jnp.bfloat16)
    ft = jnp.dot(xb, wt_ref[...], preferred_element_type=jnp.float32)
    e3 = jnp.exp(ft.reshape(H, W, K))
    s = jnp.sum(e3, axis=0, keepdims=True)
    a = (e3 / s).reshape(N, K)
    v = jax.lax.dot_general(
        a.astype(jnp.bfloat16), xb, (((0,), (0,)), ((), ())),
        preferred_element_type=jnp.float32)
    v = v - cw_ref[...]
    ss = jnp.sum(v * v, axis=0, keepdims=True)
    return v * jax.lax.rsqrt(jnp.maximum(ss, 1e-24))


def _netvlad_kernel(x_ref, wt_ref, cw_ref, o_ref, acc_ref):
    for g in range(G):
        y = _one_batch(x_ref[0, g], wt_ref, cw_ref)
        acc_ref[g] = y
    o_ref[:, 0, :, :] = jnp.transpose(acc_ref[...], (1, 0, 2))


def kernel(x, w, b_conv, c):
    del b_conv
    xn = x.transpose(0, 2, 3, 1).reshape(B // G, G, N, D)
    wt = w.T.astype(jnp.bfloat16)
    cw = c * jnp.float32(W)
    out = pl.pallas_call(
        _netvlad_kernel,
        grid=(B // G,),
        in_specs=[
            pl.BlockSpec((1, G, N, D), lambda i: (i, 0, 0, 0)),
            pl.BlockSpec((D, K), lambda i: (0, 0)),
            pl.BlockSpec((K, D), lambda i: (0, 0)),
        ],
        out_specs=pl.BlockSpec((K, 1, G, D), lambda i: (0, i, 0, 0)),
        out_shape=jax.ShapeDtypeStruct((K, B // G, G, D), jnp.float32),
        scratch_shapes=[pltpu.VMEM((G, K, D), jnp.float32)],
        compiler_params=pltpu.CompilerParams(
            dimension_semantics=("arbitrary",),
        ),
    )(xn, wt, cw)
    return jnp.transpose(out.reshape(K, B, D), (2, 0, 1))


# speedup vs baseline: 4.4355x; 1.3133x over previous
import jax
import jax.numpy as jnp
from jax.experimental import pallas as pl
from jax.experimental.pallas import tpu as pltpu

B, D, H, W, K = 64, 512, 32, 32, 64
N = H * W
G = 4
S = 8


def _netvlad_kernel(x_ref, wt_ref, cw_ref, o_ref, acc_ref):
    i = pl.program_id(0)
    j = jax.lax.rem(i, S // G) * G
    xb = x_ref[0].reshape(G * N, D).astype(jnp.bfloat16)
    ft = jnp.dot(xb, wt_ref[...], preferred_element_type=jnp.float32)
    e4 = jnp.exp2(ft.reshape(G, H, W, K))
    s = jnp.sum(e4, axis=1, keepdims=True)
    a = (e4 / s).reshape(G * N, K).astype(jnp.bfloat16)
    for g in range(G):
        v = jax.lax.dot_general(
            a[g * N:(g + 1) * N], xb[g * N:(g + 1) * N],
            (((0,), (0,)), ((), ())),
            preferred_element_type=jnp.float32)
        v = v - cw_ref[...]
        ss = jnp.sum(v * v, axis=0, keepdims=True)
        acc_ref[pl.ds(j + g, 1)] = (
            v * jax.lax.rsqrt(jnp.maximum(ss, 1e-24))).reshape(1, K, D)

    @pl.when(jax.lax.rem(i, S // G) == S // G - 1)
    def _():
        o_ref[:, 0, :, :] = jnp.transpose(acc_ref[...], (1, 0, 2))


def kernel(x, w, b_conv, c):
    del b_conv
    xn = x.transpose(0, 2, 3, 1).reshape(B // G, G, N, D)
    wt = (w.T * jnp.float32(1.4426950408889634)).astype(jnp.bfloat16)
    cw = c * jnp.float32(W)
    out = pl.pallas_call(
        _netvlad_kernel,
        grid=(B // G,),
        in_specs=[
            pl.BlockSpec((1, G, N, D), lambda i: (i, 0, 0, 0)),
            pl.BlockSpec((D, K), lambda i: (0, 0)),
            pl.BlockSpec((K, D), lambda i: (0, 0)),
        ],
        out_specs=pl.BlockSpec((K, 1, S, D), lambda i: (0, i // (S // G), 0, 0)),
        out_shape=jax.ShapeDtypeStruct((K, B // S, S, D), jnp.float32),
        scratch_shapes=[pltpu.VMEM((S, K, D), jnp.float32)],
        compiler_params=pltpu.CompilerParams(
            dimension_semantics=("arbitrary",),
            vmem_limit_bytes=64 * 1024 * 1024,
        ),
    )(xn, wt, cw)
    return jnp.transpose(out.reshape(K, B, D), (2, 0, 1))


# speedup vs baseline: 4.9284x; 1.1111x over previous
import jax
import jax.numpy as jnp
from jax.experimental import pallas as pl
from jax.experimental.pallas import tpu as pltpu

B, D, H, W, K = 64, 512, 32, 32, 64
N = H * W
G = 8
S = 8


def _netvlad_kernel(x_ref, wt_ref, cw_ref, o_ref, acc_ref):
    i = pl.program_id(0)
    j = jax.lax.rem(i, S // G) * G
    xb = x_ref[0].reshape(G * N, D).astype(jnp.bfloat16)
    ft = jnp.dot(xb, wt_ref[...], preferred_element_type=jnp.float32)
    e4 = jnp.exp2(ft.reshape(G, H, W, K))
    s = jnp.sum(e4, axis=1, keepdims=True)
    a = (e4 / s).reshape(G * N, K).astype(jnp.bfloat16)
    for g in range(G):
        v = jax.lax.dot_general(
            a[g * N:(g + 1) * N], xb[g * N:(g + 1) * N],
            (((0,), (0,)), ((), ())),
            preferred_element_type=jnp.float32)
        v = v - cw_ref[...]
        ss = jnp.sum(v * v, axis=0, keepdims=True)
        acc_ref[pl.ds(j + g, 1)] = (
            v * jax.lax.rsqrt(jnp.maximum(ss, 1e-24))).reshape(1, K, D)

    @pl.when(jax.lax.rem(i, S // G) == S // G - 1)
    def _():
        o_ref[:, 0, :, :] = jnp.transpose(acc_ref[...], (1, 0, 2))


def kernel(x, w, b_conv, c):
    del b_conv
    xn = x.transpose(0, 2, 3, 1).reshape(B // G, G, N, D)
    wt = (w.T * jnp.float32(1.4426950408889634)).astype(jnp.bfloat16)
    cw = c * jnp.float32(W)
    out = pl.pallas_call(
        _netvlad_kernel,
        grid=(B // G,),
        in_specs=[
            pl.BlockSpec((1, G, N, D), lambda i: (i, 0, 0, 0)),
            pl.BlockSpec((D, K), lambda i: (0, 0)),
            pl.BlockSpec((K, D), lambda i: (0, 0)),
        ],
        out_specs=pl.BlockSpec((K, 1, S, D), lambda i: (0, i // (S // G), 0, 0)),
        out_shape=jax.ShapeDtypeStruct((K, B // S, S, D), jnp.float32),
        scratch_shapes=[pltpu.VMEM((S, K, D), jnp.float32)],
        compiler_params=pltpu.CompilerParams(
            dimension_semantics=("arbitrary",),
            vmem_limit_bytes=64 * 1024 * 1024,
        ),
    )(xn, wt, cw)
    return jnp.transpose(out.reshape(K, B, D), (2, 0, 1))


# speedup vs baseline: 5.2473x; 1.0647x over previous
import jax
import jax.numpy as jnp
from jax.experimental import pallas as pl
from jax.experimental.pallas import tpu as pltpu

B, D, H, W, K = 64, 512, 32, 32, 64
N = H * W
G = 8
S = 8


def _netvlad_kernel(x_ref, w_ref, c_ref, o_ref, acc_ref):
    i = pl.program_id(0)
    j = jax.lax.rem(i, S // G) * G
    wb = (w_ref[...] * jnp.float32(1.4426950408889634)).astype(jnp.bfloat16)
    xb = x_ref[0].reshape(G * N, D).astype(jnp.bfloat16)
    ft = jax.lax.dot_general(
        xb, wb, (((1,), (1,)), ((), ())),
        preferred_element_type=jnp.float32)
    e4 = jnp.exp2(ft.reshape(G, H, W, K))
    s = jnp.sum(e4, axis=1, keepdims=True)
    a = (e4 / s).reshape(G * N, K).astype(jnp.bfloat16)
    for g in range(G):
        v = jax.lax.dot_general(
            a[g * N:(g + 1) * N], xb[g * N:(g + 1) * N],
            (((0,), (0,)), ((), ())),
            preferred_element_type=jnp.float32)
        v = v - c_ref[...] * jnp.float32(W)
        ss = jnp.sum(v * v, axis=0, keepdims=True)
        acc_ref[pl.ds(j + g, 1)] = (
            v * jax.lax.rsqrt(jnp.maximum(ss, 1e-24))).reshape(1, K, D)

    @pl.when(jax.lax.rem(i, S // G) == S // G - 1)
    def _():
        o_ref[:, 0, :, :] = jnp.transpose(acc_ref[...], (1, 0, 2))


def kernel(x, w, b_conv, c):
    del b_conv
    xn = x.transpose(0, 2, 3, 1).reshape(B // G, G, N, D)
    out = pl.pallas_call(
        _netvlad_kernel,
        grid=(B // G,),
        in_specs=[
            pl.BlockSpec((1, G, N, D), lambda i: (i, 0, 0, 0)),
            pl.BlockSpec((K, D), lambda i: (0, 0)),
            pl.BlockSpec((K, D), lambda i: (0, 0)),
        ],
        out_specs=pl.BlockSpec((K, 1, S, D), lambda i: (0, i // (S // G), 0, 0)),
        out_shape=jax.ShapeDtypeStruct((K, B // S, S, D), jnp.float32),
        scratch_shapes=[pltpu.VMEM((S, K, D), jnp.float32)],
        compiler_params=pltpu.CompilerParams(
            dimension_semantics=("arbitrary",),
            vmem_limit_bytes=64 * 1024 * 1024,
        ),
    )(xn, w, c)
    return jnp.transpose(out.reshape(K, B, D), (2, 0, 1))


# speedup vs baseline: 5.2502x; 1.0005x over previous
import jax
import jax.numpy as jnp
from jax.experimental import pallas as pl
from jax.experimental.pallas import tpu as pltpu

B, D, H, W, K = 64, 512, 32, 32, 64
N = H * W
G = 8
S = 8


def _netvlad_kernel(x_ref, w_ref, c_ref, o_ref, acc_ref):
    wb = (w_ref[...] * jnp.float32(1.4426950408889634)).astype(jnp.bfloat16)
    xb = x_ref[0].reshape(G * N, D).astype(jnp.bfloat16)
    ft = jax.lax.dot_general(
        xb, wb, (((1,), (1,)), ((), ())),
        preferred_element_type=jnp.float32)
    e4 = jnp.exp2(ft.reshape(G, H, W, K))
    s = jnp.sum(e4, axis=1, keepdims=True)
    a = (e4 / s).reshape(G * N, K).astype(jnp.bfloat16)
    for g in range(G):
        v = jax.lax.dot_general(
            a[g * N:(g + 1) * N], xb[g * N:(g + 1) * N],
            (((0,), (0,)), ((), ())),
            preferred_element_type=jnp.float32)
        v = v - c_ref[...] * jnp.float32(W)
        ss = jnp.sum(v * v, axis=0, keepdims=True)
        acc_ref[g] = v * jax.lax.rsqrt(jnp.maximum(ss, 1e-24))
    o_ref[:, 0, :, :] = jnp.transpose(acc_ref[...], (1, 0, 2))


def kernel(x, w, b_conv, c):
    del b_conv
    xn = x.transpose(0, 2, 3, 1).reshape(B // G, G, N, D)
    out = pl.pallas_call(
        _netvlad_kernel,
        grid=(B // G,),
        in_specs=[
            pl.BlockSpec((1, G, N, D), lambda i: (i, 0, 0, 0)),
            pl.BlockSpec((K, D), lambda i: (0, 0)),
            pl.BlockSpec((K, D), lambda i: (0, 0)),
        ],
        out_specs=pl.BlockSpec((K, 1, S, D), lambda i: (0, i, 0, 0)),
        out_shape=jax.ShapeDtypeStruct((K, B // S, S, D), jnp.float32),
        scratch_shapes=[pltpu.VMEM((S, K, D), jnp.float32)],
        compiler_params=pltpu.CompilerParams(
            dimension_semantics=("arbitrary",),
            vmem_limit_bytes=64 * 1024 * 1024,
        ),
    )(xn, w, c)
    return jnp.transpose(out.reshape(K, B, D), (2, 0, 1))
